# Initial kernel scaffold; baseline (speedup 1.0000x reference)
import functools

import jax
import jax.numpy as jnp
from jax import lax
from jax.experimental import pallas as pl
from jax.experimental.pallas import tpu as pltpu
from jax.experimental.pallas import tpu_sc as plsc


def _gather_call(x_flat, w_flat, num_feat, rows_per_w, chunk):
    n_chunks = rows_per_w // chunk
    total_rows = x_flat.shape[0]
    d = w_flat.shape[1]
    vocab = w_flat.shape[0] // num_feat
    lanes = 16

    mesh = plsc.VectorSubcoreMesh(core_axis_name="c", subcore_axis_name="s")

    @functools.partial(
        pl.kernel,
        mesh=mesh,
        out_type=jax.ShapeDtypeStruct((total_rows, d), jnp.float32),
        scratch_types=[
            pltpu.VMEM((chunk,), jnp.int32),
            pltpu.VMEM((chunk,), jnp.int32),
            pltpu.VMEM((chunk, d), jnp.float32),
            pltpu.SemaphoreType.DMA,
        ],
    )
    def k(x_hbm, w_hbm, out_hbm, idx_v, offs_v, rows_v, sem):
        wid = lax.axis_index("s") * 2 + lax.axis_index("c")
        wbase = wid * rows_per_w

        def fill_offs(i, _):
            sl = pl.ds(i * lanes, lanes)
            v = lax.iota(jnp.int32, lanes) + i * lanes
            offs_v[sl] = lax.rem(v, num_feat) * vocab
            return 0

        lax.fori_loop(0, chunk // lanes, fill_offs, 0)

        def do_chunk(c, _):
            base = wbase + c * chunk
            pltpu.sync_copy(x_hbm.at[pl.ds(base, chunk)], idx_v)

            def add_offs(i, _):
                sl = pl.ds(i * lanes, lanes)
                idx_v[sl] = idx_v[sl] + offs_v[sl]
                return 0

            lax.fori_loop(0, chunk // lanes, add_offs, 0)
            pltpu.async_copy(w_hbm.at[idx_v], rows_v, sem).wait()
            pltpu.sync_copy(rows_v, out_hbm.at[pl.ds(base, chunk)])
            return 0

        lax.fori_loop(0, n_chunks, do_chunk, 0)

    return k(x_flat, w_flat)


def kernel(x, W):
    num_feat, vocab, d = W.shape
    batch = x.shape[0]
    total_rows = batch * num_feat

    nw = 32
    rows_per_w = total_rows // nw
    chunk = 1664

    x_flat = x.reshape(total_rows).astype(jnp.int32)
    w_flat = W.reshape(num_feat * vocab, d)
    out = _gather_call(x_flat, w_flat, num_feat, rows_per_w, chunk)
    return out.reshape(batch, num_feat, d)


# speedup vs baseline: 1.1454x; 1.1454x over previous
import functools

import jax
import jax.numpy as jnp
from jax import lax
from jax.experimental import pallas as pl
from jax.experimental.pallas import tpu as pltpu
from jax.experimental.pallas import tpu_sc as plsc


def _gather_call(x_flat, w_flat, num_feat, rows_per_w, chunk):
    n_chunks = rows_per_w // chunk
    total_rows = x_flat.shape[0]
    d = w_flat.shape[1]
    vocab = w_flat.shape[0] // num_feat
    lanes = 16

    mesh = plsc.VectorSubcoreMesh(core_axis_name="c", subcore_axis_name="s")

    @functools.partial(
        pl.kernel,
        mesh=mesh,
        compiler_params=pltpu.CompilerParams(use_tc_tiling_on_sc=False),
        out_type=jax.ShapeDtypeStruct((total_rows, d), jnp.float32),
        scratch_types=[
            pltpu.VMEM((chunk,), jnp.int32),
            pltpu.VMEM((chunk,), jnp.int32),
            pltpu.VMEM((chunk, d), jnp.float32),
            pltpu.SemaphoreType.DMA,
        ],
    )
    def k(x_hbm, w_hbm, out_hbm, idx_v, offs_v, rows_v, sem):
        wid = lax.axis_index("s") * 2 + lax.axis_index("c")
        wbase = wid * rows_per_w

        def fill_offs(i, _):
            sl = pl.ds(i * lanes, lanes)
            v = lax.iota(jnp.int32, lanes) + i * lanes
            offs_v[sl] = lax.rem(v, num_feat) * vocab
            return 0

        lax.fori_loop(0, chunk // lanes, fill_offs, 0)

        def do_chunk(c, _):
            base = wbase + c * chunk
            pltpu.sync_copy(x_hbm.at[pl.ds(base, chunk)], idx_v)

            def add_offs(i, _):
                sl = pl.ds(i * lanes, lanes)
                idx_v[sl] = idx_v[sl] + offs_v[sl]
                return 0

            lax.fori_loop(0, chunk // lanes, add_offs, 0)
            pltpu.async_copy(w_hbm.at[idx_v], rows_v, sem).wait()
            pltpu.sync_copy(rows_v, out_hbm.at[pl.ds(base, chunk)])
            return 0

        lax.fori_loop(0, n_chunks, do_chunk, 0)

    return k(x_flat, w_flat)


def kernel(x, W):
    num_feat, vocab, d = W.shape
    batch = x.shape[0]
    total_rows = batch * num_feat

    nw = 32
    rows_per_w = total_rows // nw
    chunk = 1664

    x_flat = x.reshape(total_rows).astype(jnp.int32)
    w_flat = W.reshape(num_feat * vocab, d)
    out = _gather_call(x_flat, w_flat, num_feat, rows_per_w, chunk)
    return out.reshape(batch, num_feat, d)


# speedup vs baseline: 1.1489x; 1.0031x over previous
import functools

import jax
import jax.numpy as jnp
from jax import lax
from jax.experimental import pallas as pl
from jax.experimental.pallas import tpu as pltpu
from jax.experimental.pallas import tpu_sc as plsc


def _gather_call(x_flat, w_flat, num_feat, rows_per_w, chunk):
    n_chunks = rows_per_w // chunk
    total_rows = x_flat.shape[0]
    d = w_flat.shape[1]
    vocab = w_flat.shape[0] // num_feat
    lanes = 16

    mesh = plsc.VectorSubcoreMesh(core_axis_name="c", subcore_axis_name="s")

    @functools.partial(
        pl.kernel,
        mesh=mesh,
        compiler_params=pltpu.CompilerParams(use_tc_tiling_on_sc=False),
        out_type=jax.ShapeDtypeStruct((total_rows, d), jnp.float32),
        scratch_types=[
            pltpu.VMEM((chunk,), jnp.int32),
            pltpu.VMEM((chunk,), jnp.int32),
            pltpu.VMEM((chunk,), jnp.int32),
            pltpu.VMEM((chunk, d), jnp.float32),
            pltpu.VMEM((chunk, d), jnp.float32),
            pltpu.SemaphoreType.DMA,
            pltpu.SemaphoreType.DMA,
            pltpu.SemaphoreType.DMA,
            pltpu.SemaphoreType.DMA,
            pltpu.SemaphoreType.DMA,
            pltpu.SemaphoreType.DMA,
        ],
    )
    def k(x_hbm, w_hbm, out_hbm, idx0, idx1, offs_v, rows0, rows1,
          semi0, semi1, semg0, semg1, semo0, semo1):
        wid = lax.axis_index("s") * 2 + lax.axis_index("c")
        wbase = wid * rows_per_w
        idx_b = (idx0, idx1)
        rows_b = (rows0, rows1)
        semi = (semi0, semi1)
        semg = (semg0, semg1)
        semo = (semo0, semo1)

        def fill_offs(i, _):
            sl = pl.ds(i * lanes, lanes)
            v = lax.iota(jnp.int32, lanes) + i * lanes
            offs_v[sl] = lax.rem(v, num_feat) * vocab
            return 0

        lax.fori_loop(0, chunk // lanes, fill_offs, 0)

        def row_slice(c):
            return pl.ds(wbase + c * chunk, chunk)

        idx_d = [None] * n_chunks
        out_d = [None] * n_chunks
        idx_d[0] = pltpu.async_copy(x_hbm.at[row_slice(0)], idx_b[0], semi[0])
        for c in range(n_chunks):
            b = c % 2
            idx_v, rows_v = idx_b[b], rows_b[b]
            idx_d[c].wait()
            if c + 1 < n_chunks:
                nb = (c + 1) % 2
                idx_d[c + 1] = pltpu.async_copy(
                    x_hbm.at[row_slice(c + 1)], idx_b[nb], semi[nb])

            def add_offs(i, _):
                sl = pl.ds(i * lanes, lanes)
                idx_v[sl] = idx_v[sl] + offs_v[sl]
                return 0

            lax.fori_loop(0, chunk // lanes, add_offs, 0)
            if c >= 2:
                out_d[c - 2].wait()
            pltpu.async_copy(w_hbm.at[idx_v], rows_v, semg[b]).wait()
            out_d[c] = pltpu.async_copy(rows_v, out_hbm.at[row_slice(c)], semo[b])
        out_d[n_chunks - 2].wait()
        out_d[n_chunks - 1].wait()

    return k(x_flat, w_flat)


def kernel(x, W):
    num_feat, vocab, d = W.shape
    batch = x.shape[0]
    total_rows = batch * num_feat

    nw = 32
    rows_per_w = total_rows // nw
    chunk = 1664

    x_flat = x.reshape(total_rows).astype(jnp.int32)
    w_flat = W.reshape(num_feat * vocab, d)
    out = _gather_call(x_flat, w_flat, num_feat, rows_per_w, chunk)
    return out.reshape(batch, num_feat, d)


# speedup vs baseline: 1.1541x; 1.0045x over previous
import functools

import jax
import jax.numpy as jnp
from jax import lax
from jax.experimental import pallas as pl
from jax.experimental.pallas import tpu as pltpu
from jax.experimental.pallas import tpu_sc as plsc

_NBUF = 4
_GDEPTH = 2


def _gather_call(x_flat, w_flat, num_feat, rows_per_w, chunk):
    n_chunks = rows_per_w // chunk
    total_rows = x_flat.shape[0]
    d = w_flat.shape[1]
    vocab = w_flat.shape[0] // num_feat
    lanes = 16

    mesh = plsc.VectorSubcoreMesh(core_axis_name="c", subcore_axis_name="s")

    @functools.partial(
        pl.kernel,
        mesh=mesh,
        compiler_params=pltpu.CompilerParams(use_tc_tiling_on_sc=False),
        out_type=jax.ShapeDtypeStruct((total_rows, d), jnp.float32),
        scratch_types=(
            [pltpu.VMEM((chunk,), jnp.int32) for _ in range(_NBUF)]
            + [pltpu.VMEM((chunk,), jnp.int32)]
            + [pltpu.VMEM((chunk, d), jnp.float32) for _ in range(_NBUF)]
            + [pltpu.SemaphoreType.DMA for _ in range(3 * _NBUF)]
        ),
    )
    def k(x_hbm, w_hbm, out_hbm, *refs):
        idx_b = refs[0:_NBUF]
        offs_v = refs[_NBUF]
        rows_b = refs[_NBUF + 1:2 * _NBUF + 1]
        sems = refs[2 * _NBUF + 1:]
        semi = sems[0:_NBUF]
        semg = sems[_NBUF:2 * _NBUF]
        semo = sems[2 * _NBUF:3 * _NBUF]

        wid = lax.axis_index("s") * 2 + lax.axis_index("c")
        wbase = wid * rows_per_w

        def fill_offs(i, _):
            sl = pl.ds(i * lanes, lanes)
            v = lax.iota(jnp.int32, lanes) + i * lanes
            offs_v[sl] = lax.rem(v, num_feat) * vocab
            return 0

        lax.fori_loop(0, chunk // lanes, fill_offs, 0)

        def row_slice(c):
            return pl.ds(wbase + c * chunk, chunk)

        idx_d = [None] * n_chunks
        gat_d = [None] * n_chunks
        out_d = [None] * n_chunks
        idx_d[0] = pltpu.async_copy(x_hbm.at[row_slice(0)], idx_b[0], semi[0])
        for c in range(n_chunks):
            b = c % _NBUF
            idx_v, rows_v = idx_b[b], rows_b[b]
            idx_d[c].wait()
            if c + 1 < n_chunks:
                nb = (c + 1) % _NBUF
                idx_d[c + 1] = pltpu.async_copy(
                    x_hbm.at[row_slice(c + 1)], idx_b[nb], semi[nb])

            def add_offs(i, _):
                sl = pl.ds(i * lanes, lanes)
                idx_v[sl] = idx_v[sl] + offs_v[sl]
                return 0

            lax.fori_loop(0, chunk // lanes, add_offs, 0)
            if c >= _NBUF:
                out_d[c - _NBUF].wait()
            gat_d[c] = pltpu.async_copy(w_hbm.at[idx_v], rows_v, semg[b])
            if c >= _GDEPTH:
                w_ = c - _GDEPTH
                gat_d[w_].wait()
                out_d[w_] = pltpu.async_copy(
                    rows_b[w_ % _NBUF], out_hbm.at[row_slice(w_)], semo[w_ % _NBUF])
        for w_ in range(n_chunks - _GDEPTH, n_chunks):
            gat_d[w_].wait()
            out_d[w_] = pltpu.async_copy(
                rows_b[w_ % _NBUF], out_hbm.at[row_slice(w_)], semo[w_ % _NBUF])
        for w_ in range(max(0, n_chunks - _NBUF), n_chunks):
            out_d[w_].wait()

    return k(x_flat, w_flat)


def kernel(x, W):
    num_feat, vocab, d = W.shape
    batch = x.shape[0]
    total_rows = batch * num_feat

    nw = 32
    rows_per_w = total_rows // nw
    chunk = 832

    x_flat = x.reshape(total_rows).astype(jnp.int32)
    w_flat = W.reshape(num_feat * vocab, d)
    out = _gather_call(x_flat, w_flat, num_feat, rows_per_w, chunk)
    return out.reshape(batch, num_feat, d)


# speedup vs baseline: 1.1551x; 1.0009x over previous
import functools

import jax
import jax.numpy as jnp
from jax import lax
from jax.experimental import pallas as pl
from jax.experimental.pallas import tpu as pltpu
from jax.experimental.pallas import tpu_sc as plsc


def _gather_call(x_flat, w_flat, num_feat, rows_per_w, chunk):
    n_chunks = rows_per_w // chunk
    total_rows = x_flat.shape[0]
    d = w_flat.shape[1]
    vocab = w_flat.shape[0] // num_feat
    lanes = 16

    mesh = plsc.VectorSubcoreMesh(core_axis_name="c", subcore_axis_name="s")

    @functools.partial(
        pl.kernel,
        mesh=mesh,
        compiler_params=pltpu.CompilerParams(use_tc_tiling_on_sc=False),
        out_type=jax.ShapeDtypeStruct((total_rows, d), jnp.float32),
        scratch_types=(
            [pltpu.VMEM((chunk,), jnp.int32) for _ in range(2)]
            + [pltpu.VMEM((chunk,), jnp.int32)]
            + [pltpu.VMEM((chunk, d), jnp.float32) for _ in range(2)]
            + [pltpu.SemaphoreType.DMA for _ in range(6)]
        ),
    )
    def k(x_hbm, w_hbm, out_hbm, idx0, idx1, offs_v, rows0, rows1,
          semi0, semi1, semg0, semg1, semo0, semo1):
        idx_b = (idx0, idx1)
        rows_b = (rows0, rows1)
        semi = (semi0, semi1)
        semg = (semg0, semg1)
        semo = (semo0, semo1)

        wid = lax.axis_index("s") * 2 + lax.axis_index("c")
        wbase = wid * rows_per_w

        def fill_offs(i, _):
            sl = pl.ds(i * lanes, lanes)
            v = lax.iota(jnp.int32, lanes) + i * lanes
            offs_v[sl] = lax.rem(v, num_feat) * vocab
            return 0

        lax.fori_loop(0, chunk // lanes, fill_offs, 0)

        def row_slice(c):
            return pl.ds(wbase + c * chunk, chunk)

        def enqueue_gathers(b):
            idx_v, rows_v = idx_b[b], rows_b[b]

            def body(g, _):
                sl = pl.ds(g * lanes, lanes)
                v = idx_v[sl] + offs_v[sl]
                pltpu.async_copy(w_hbm.at[v], rows_v.at[sl], semg[b])
                return 0

            lax.fori_loop(0, chunk // lanes, body, 0)

        def drain_gathers(b):
            pltpu.make_async_copy(
                w_hbm.at[pl.ds(0, chunk)], rows_b[b], semg[b]).wait()

        idx_d = [None] * n_chunks
        out_d = [None] * n_chunks
        idx_d[0] = pltpu.async_copy(x_hbm.at[row_slice(0)], idx_b[0], semi[0])
        for c in range(n_chunks):
            b = c % 2
            idx_d[c].wait()
            if c + 1 < n_chunks:
                nb = (c + 1) % 2
                idx_d[c + 1] = pltpu.async_copy(
                    x_hbm.at[row_slice(c + 1)], idx_b[nb], semi[nb])
            if c >= 2:
                out_d[c - 2].wait()
            enqueue_gathers(b)
            if c >= 1:
                drain_gathers(1 - b)
                out_d[c - 1] = pltpu.async_copy(
                    rows_b[1 - b], out_hbm.at[row_slice(c - 1)], semo[1 - b])
        last_b = (n_chunks - 1) % 2
        drain_gathers(last_b)
        out_d[n_chunks - 1] = pltpu.async_copy(
            rows_b[last_b], out_hbm.at[row_slice(n_chunks - 1)], semo[last_b])
        out_d[n_chunks - 2].wait()
        out_d[n_chunks - 1].wait()

    return k(x_flat, w_flat)


def kernel(x, W):
    num_feat, vocab, d = W.shape
    batch = x.shape[0]
    total_rows = batch * num_feat

    nw = 32
    rows_per_w = total_rows // nw
    chunk = 1664

    x_flat = x.reshape(total_rows).astype(jnp.int32)
    w_flat = W.reshape(num_feat * vocab, d)
    out = _gather_call(x_flat, w_flat, num_feat, rows_per_w, chunk)
    return out.reshape(batch, num_feat, d)
